# baseline (speedup 1.0000x reference)
import functools

import jax
import jax.numpy as jnp
from jax import lax
from jax.experimental import pallas as pl
from jax.experimental.pallas import tpu as pltpu
from jax.experimental.pallas import tpu_sc as plsc

_NC = 2
_NS = 16
_L = 16
_NW = _NC * _NS
_CHUNK = 3136
_NPAD = _NW * _CHUNK
_UNROLL = 4

_P = (
    0.9999998807907104,
    -0.5,
    0.3333473205566406,
    -0.2500125467777252,
    0.19944770634174347,
    -0.16575729846954346,
    0.15056419372558594,
    -0.14296768605709076,
    0.08383616805076599,
)
_LN2_HI = 0.69314575
_LN2_LO = 1.4286068e-06
_EXP_SHIFT = 0x3F800000 - 0x3F3504F3
_MANT_MASK = 0x007FFFFF
_MANT_BASE = 0x3F3504F3


def _vlog(x):
    ix = lax.bitcast_convert_type(x, jnp.int32)
    ix = ix + jnp.int32(_EXP_SHIFT)
    k = (ix >> 23) - jnp.int32(127)
    m = lax.bitcast_convert_type(
        (ix & jnp.int32(_MANT_MASK)) + jnp.int32(_MANT_BASE), jnp.float32)
    f = m - jnp.float32(1.0)
    p = jnp.float32(_P[8])
    for c in _P[7::-1]:
        p = p * f + jnp.float32(c)
    p = p * f
    kf = k.astype(jnp.float32)
    return (kf * jnp.float32(_LN2_HI) + p) + kf * jnp.float32(_LN2_LO)


@functools.partial(
    pl.kernel,
    mesh=plsc.VectorSubcoreMesh(core_axis_name="c", subcore_axis_name="s"),
    out_type=jax.ShapeDtypeStruct((_NPAD,), jnp.int32),
    scratch_types=[
        pltpu.VMEM((_CHUNK,), jnp.float32),
        pltpu.VMEM((_CHUNK,), jnp.float32),
        pltpu.VMEM((_CHUNK,), jnp.int32),
        pltpu.VMEM((2 * _L,), jnp.float32),
    ],
)
def _sc_select(e_hbm, g_hbm, thr_hbm, out_hbm, e_v, g_v, o_v, thr_v):
    wid = lax.axis_index("s") * _NC + lax.axis_index("c")
    base = wid * _CHUNK
    pltpu.sync_copy(thr_hbm, thr_v)
    pltpu.sync_copy(e_hbm.at[pl.ds(base, _CHUNK)], e_v)
    pltpu.sync_copy(g_hbm.at[pl.ds(base, _CHUNK)], g_v)

    ht = thr_v[pl.ds(0, _L)]
    lt = thr_v[pl.ds(_L, _L)]
    two = jnp.full((_L,), 2, jnp.int32)
    one = jnp.full((_L,), 1, jnp.int32)
    zero = jnp.full((_L,), 0, jnp.int32)

    def step(i, carry):
        for u in range(_UNROLL):
            sl = pl.ds((i * _UNROLL + u) * _L, _L)
            e = e_v[sl]
            g = g_v[sl]
            log_e = _vlog(jnp.maximum(e, jnp.float32(0.1)))
            ne = jnp.clip((log_e + jnp.float32(1.0)) * jnp.float32(1.0 / 3.5),
                          jnp.float32(0.0), jnp.float32(1.0))
            log_g = _vlog(jnp.maximum(g, jnp.float32(1e-4)))
            ng = jnp.clip((log_g + jnp.float32(9.0)) * jnp.float32(1.0 / 8.3),
                          jnp.float32(0.0), jnp.float32(1.0))
            score = (jnp.float32(0.5) * ne
                     + jnp.float32(0.5) * (jnp.float32(1.0) - ng))
            o_v[sl] = jnp.where(score > ht, two,
                                jnp.where(score < lt, one, zero))
        return carry

    lax.fori_loop(0, _CHUNK // (_L * _UNROLL), step, 0)
    pltpu.sync_copy(o_v, out_hbm.at[pl.ds(base, _CHUNK)])


def kernel(expansion, fiedler_gradient_mag, high_threshold, low_threshold):
    n = expansion.shape[0]
    pad = _NPAD - n
    e = jnp.pad(expansion, (0, pad), constant_values=1.0)
    g = jnp.pad(fiedler_gradient_mag, (0, pad), constant_values=1.0)
    thr = jnp.concatenate([
        jnp.full((_L,), high_threshold, jnp.float32),
        jnp.full((_L,), low_threshold, jnp.float32),
    ])
    out = _sc_select(e, g, thr)
    return out[:n]


# speedup vs baseline: 1.1428x; 1.1428x over previous
import functools

import jax
import jax.numpy as jnp
from jax import lax
from jax.experimental import pallas as pl
from jax.experimental.pallas import tpu as pltpu
from jax.experimental.pallas import tpu_sc as plsc

_NC = 2
_NS = 16
_L = 16
_NW = _NC * _NS
_N = 100000
_CHUNK = 3136
_UNROLL = 4

_P = (
    0.0,
    0.9999998807907104,
    -0.5,
    0.3333473205566406,
    -0.2500125467777252,
    0.19944770634174347,
    -0.16575729846954346,
    0.15056419372558594,
    -0.14296768605709076,
    0.08383616805076599,
)
_LN2 = 0.6931471805599453
_EXP_SHIFT = 0x3F800000 - 0x3F3504F3
_MANT_MASK = 0x007FFFFF
_MANT_BASE = 0x3F3504F3

_QE = tuple((c + (1.0 if j == 0 else 0.0)) / 3.5 for j, c in enumerate(_P))
_QG = tuple((c + (9.0 if j == 0 else 0.0)) / 8.3 for j, c in enumerate(_P))
_KE = _LN2 / 3.5
_KG = _LN2 / 8.3


def _split(x):
    ix = lax.bitcast_convert_type(x, jnp.int32) + jnp.int32(_EXP_SHIFT)
    k = (ix >> 23) - jnp.int32(127)
    m = lax.bitcast_convert_type(
        (ix & jnp.int32(_MANT_MASK)) + jnp.int32(_MANT_BASE), jnp.float32)
    return k.astype(jnp.float32), m - jnp.float32(1.0)


def _norm(kf, f, q, kscale):
    p = jnp.float32(q[9])
    for c in q[8::-1]:
        p = p * f + jnp.float32(c)
    return jnp.clip(kf * jnp.float32(kscale) + p,
                    jnp.float32(0.0), jnp.float32(1.0))


@functools.partial(
    pl.kernel,
    mesh=plsc.VectorSubcoreMesh(core_axis_name="c", subcore_axis_name="s"),
    out_type=jax.ShapeDtypeStruct((_N,), jnp.int32),
    scratch_types=[
        pltpu.VMEM((_CHUNK,), jnp.float32),
        pltpu.VMEM((_CHUNK,), jnp.float32),
        pltpu.VMEM((_CHUNK,), jnp.int32),
        pltpu.VMEM((2 * _L,), jnp.float32),
        pltpu.SemaphoreType.DMA,
        pltpu.SemaphoreType.DMA,
        pltpu.SemaphoreType.DMA,
    ],
)
def _sc_select(e_hbm, g_hbm, thr_hbm, out_hbm, e_v, g_v, o_v, thr_v,
               sem_e, sem_g, sem_t):
    wid = lax.axis_index("s") * _NC + lax.axis_index("c")
    base = jnp.minimum(wid * _CHUNK, _N - _CHUNK)
    ce = pltpu.async_copy(e_hbm.at[pl.ds(base, _CHUNK)], e_v, sem_e)
    cg = pltpu.async_copy(g_hbm.at[pl.ds(base, _CHUNK)], g_v, sem_g)
    ct = pltpu.async_copy(thr_hbm, thr_v, sem_t)
    ct.wait()
    tht = thr_v[pl.ds(0, _L)]
    tlt = thr_v[pl.ds(_L, _L)]
    two = jnp.full((_L,), 2, jnp.int32)
    one = jnp.full((_L,), 1, jnp.int32)
    zero = jnp.full((_L,), 0, jnp.int32)
    ce.wait()
    cg.wait()

    def step(i, carry):
        for u in range(_UNROLL):
            sl = pl.ds((i * _UNROLL + u) * _L, _L)
            ke, fe = _split(jnp.maximum(e_v[sl], jnp.float32(0.1)))
            kg, fg = _split(jnp.maximum(g_v[sl], jnp.float32(1e-4)))
            d = _norm(ke, fe, _QE, _KE) - _norm(kg, fg, _QG, _KG)
            o_v[sl] = jnp.where(d > tht, two, jnp.where(d < tlt, one, zero))
        return carry

    lax.fori_loop(0, _CHUNK // (_L * _UNROLL), step, 0)
    pltpu.sync_copy(o_v, out_hbm.at[pl.ds(base, _CHUNK)])


def kernel(expansion, fiedler_gradient_mag, high_threshold, low_threshold):
    thr = jnp.concatenate([
        jnp.full((_L,), 2.0 * high_threshold - 1.0, jnp.float32),
        jnp.full((_L,), 2.0 * low_threshold - 1.0, jnp.float32),
    ])
    return _sc_select(expansion, fiedler_gradient_mag, thr)


# speedup vs baseline: 1.1620x; 1.0168x over previous
import functools

import jax
import jax.numpy as jnp
from jax import lax
from jax.experimental import pallas as pl
from jax.experimental.pallas import tpu as pltpu
from jax.experimental.pallas import tpu_sc as plsc

_NC = 2
_NS = 16
_L = 16
_NW = _NC * _NS
_N = 100000
_CHUNK = 3136
_UNROLL = 1

_P = (
    0.0,
    0.9999998807907104,
    -0.5,
    0.3333473205566406,
    -0.2500125467777252,
    0.19944770634174347,
    -0.16575729846954346,
    0.15056419372558594,
    -0.14296768605709076,
    0.08383616805076599,
)
_LN2 = 0.6931471805599453
_EXP_SHIFT = 0x3F800000 - 0x3F3504F3
_MANT_MASK = 0x007FFFFF
_MANT_BASE = 0x3F3504F3

_QE = tuple((c + (1.0 if j == 0 else 0.0)) / 3.5 for j, c in enumerate(_P))
_QG = tuple((c + (9.0 if j == 0 else 0.0)) / 8.3 for j, c in enumerate(_P))
_KE = _LN2 / 3.5
_KG = _LN2 / 8.3


def _split(x):
    ix = lax.bitcast_convert_type(x, jnp.int32) + jnp.int32(_EXP_SHIFT)
    k = (ix >> 23) - jnp.int32(127)
    m = lax.bitcast_convert_type(
        (ix & jnp.int32(_MANT_MASK)) + jnp.int32(_MANT_BASE), jnp.float32)
    return k.astype(jnp.float32), m - jnp.float32(1.0)


def _norm(kf, f, q, kscale):
    p = jnp.float32(q[9])
    for c in q[8::-1]:
        p = p * f + jnp.float32(c)
    return jnp.clip(kf * jnp.float32(kscale) + p,
                    jnp.float32(0.0), jnp.float32(1.0))


@functools.partial(
    pl.kernel,
    mesh=plsc.VectorSubcoreMesh(core_axis_name="c", subcore_axis_name="s"),
    out_type=jax.ShapeDtypeStruct((_N,), jnp.int32),
    scratch_types=[
        pltpu.VMEM((_CHUNK,), jnp.float32),
        pltpu.VMEM((_CHUNK,), jnp.float32),
        pltpu.VMEM((_CHUNK,), jnp.int32),
        pltpu.VMEM((2 * _L,), jnp.float32),
        pltpu.SemaphoreType.DMA,
        pltpu.SemaphoreType.DMA,
        pltpu.SemaphoreType.DMA,
    ],
)
def _sc_select(e_hbm, g_hbm, thr_hbm, out_hbm, e_v, g_v, o_v, thr_v,
               sem_e, sem_g, sem_t):
    wid = lax.axis_index("s") * _NC + lax.axis_index("c")
    base = jnp.minimum(wid * _CHUNK, _N - _CHUNK)
    ce = pltpu.async_copy(e_hbm.at[pl.ds(base, _CHUNK)], e_v, sem_e)
    cg = pltpu.async_copy(g_hbm.at[pl.ds(base, _CHUNK)], g_v, sem_g)
    ct = pltpu.async_copy(thr_hbm, thr_v, sem_t)
    ct.wait()
    tht = thr_v[pl.ds(0, _L)]
    tlt = thr_v[pl.ds(_L, _L)]
    two = jnp.full((_L,), 2, jnp.int32)
    one = jnp.full((_L,), 1, jnp.int32)
    zero = jnp.full((_L,), 0, jnp.int32)
    ce.wait()
    cg.wait()

    def step(i, carry):
        for u in range(_UNROLL):
            sl = pl.ds((i * _UNROLL + u) * _L, _L)
            ke, fe = _split(jnp.maximum(e_v[sl], jnp.float32(0.1)))
            kg, fg = _split(jnp.maximum(g_v[sl], jnp.float32(1e-4)))
            d = _norm(ke, fe, _QE, _KE) - _norm(kg, fg, _QG, _KG)
            o_v[sl] = jnp.where(d > tht, two, jnp.where(d < tlt, one, zero))
        return carry

    lax.fori_loop(0, _CHUNK // (_L * _UNROLL), step, 0)
    pltpu.sync_copy(o_v, out_hbm.at[pl.ds(base, _CHUNK)])


def kernel(expansion, fiedler_gradient_mag, high_threshold, low_threshold):
    thr = jnp.concatenate([
        jnp.full((_L,), 2.0 * high_threshold - 1.0, jnp.float32),
        jnp.full((_L,), 2.0 * low_threshold - 1.0, jnp.float32),
    ])
    return _sc_select(expansion, fiedler_gradient_mag, thr)
